# Initial kernel scaffold; baseline (speedup 1.0000x reference)
import jax
import jax.numpy as jnp
from jax.experimental import pallas as pl


def kernel(x, W_enc, b_enc, W_dec, b_pre, k):
    raise NotImplementedError("write your pallas kernel here")


# speedup vs baseline: 13.2821x; 13.2821x over previous
import functools

import jax
import jax.numpy as jnp
from jax import lax
from jax.experimental import pallas as pl
from jax.experimental.pallas import tpu as pltpu
from jax.experimental.pallas import tpu_sc as plsc

BATCH = 8192
DM = 2048
DS = 16384
TOPK = 64

NEGINF = float("-inf")


_ENC_BM = 1024
_ENC_BN = 512


def _enc_body(x_ref, bpre_ref, w_ref, benc_ref, o_ref):
    xc = x_ref[...] - bpre_ref[...]
    acc = lax.dot_general(
        xc, w_ref[...], (((1,), (0,)), ((), ())),
        preferred_element_type=jnp.float32)
    o_ref[...] = acc + benc_ref[...]


def _encode(x, W_enc, b_enc, b_pre):
    grid = (BATCH // _ENC_BM, DS // _ENC_BN)
    return pl.pallas_call(
        _enc_body,
        grid=grid,
        in_specs=[
            pl.BlockSpec((_ENC_BM, DM), lambda i, j: (i, 0)),
            pl.BlockSpec((1, DM), lambda i, j: (0, 0)),
            pl.BlockSpec((DM, _ENC_BN), lambda i, j: (0, j)),
            pl.BlockSpec((1, _ENC_BN), lambda i, j: (0, j)),
        ],
        out_specs=pl.BlockSpec((_ENC_BM, _ENC_BN), lambda i, j: (i, j)),
        out_shape=jax.ShapeDtypeStruct((BATCH, DS), jnp.float32),
        compiler_params=pltpu.CompilerParams(
            dimension_semantics=("parallel", "parallel")),
    )(x, b_pre.reshape(1, DM), W_enc, b_enc.reshape(1, DS))


_DEC_BM = 512
_DEC_BK = 2048


def _dec_body(l_ref, w_ref, bpre_ref, o_ref):
    kk = pl.program_id(1)
    acc = lax.dot_general(
        l_ref[...].astype(jnp.bfloat16), w_ref[...],
        (((1,), (0,)), ((), ())),
        preferred_element_type=jnp.float32)

    @pl.when(kk == 0)
    def _():
        o_ref[...] = acc + bpre_ref[...]

    @pl.when(kk != 0)
    def _():
        o_ref[...] = o_ref[...] + acc


def _decode(dense, W_dec_bf16, b_pre):
    grid = (BATCH // _DEC_BM, DS // _DEC_BK)
    return pl.pallas_call(
        _dec_body,
        grid=grid,
        in_specs=[
            pl.BlockSpec((_DEC_BM, _DEC_BK), lambda i, kk: (i, kk)),
            pl.BlockSpec((_DEC_BK, DM), lambda i, kk: (kk, 0)),
            pl.BlockSpec((1, DM), lambda i, kk: (0, 0)),
        ],
        out_specs=pl.BlockSpec((_DEC_BM, DM), lambda i, kk: (i, 0)),
        out_shape=jax.ShapeDtypeStruct((BATCH, DM), jnp.float32),
        compiler_params=pltpu.CompilerParams(
            dimension_semantics=("parallel", "arbitrary")),
    )(dense, W_dec_bf16, b_pre.reshape(1, DM))


_NW = 32
_RPW = BATCH // _NW
_NSUP = 64


def _scal(v):
    return v[0] if getattr(v, "ndim", 0) else v


def _rmax(v):
    return plsc.cummax(v)[15]


def _sc_topk(latents):
    mesh = plsc.VectorSubcoreMesh(core_axis_name="c", subcore_axis_name="s")

    @functools.partial(
        pl.kernel,
        out_type=(
            jax.ShapeDtypeStruct((BATCH, TOPK), jnp.float32),
            jax.ShapeDtypeStruct((BATCH, TOPK), jnp.int32),
            jax.ShapeDtypeStruct((BATCH, DS), jnp.float32),
        ),
        mesh=mesh,
        scratch_types=[
            pltpu.VMEM((DS,), jnp.float32),
            pltpu.VMEM((DS,), jnp.float32),
            pltpu.VMEM((DS,), jnp.float32),
            pltpu.VMEM((_NSUP * 16,), jnp.float32),
            pltpu.VMEM((_NSUP,), jnp.float32),
            pltpu.VMEM((_RPW, TOPK), jnp.float32),
            pltpu.VMEM((_RPW, TOPK), jnp.int32),
            pltpu.SemaphoreType.DMA,
            pltpu.SemaphoreType.DMA,
        ],
        compiler_params=pltpu.CompilerParams(needs_layout_passes=False),
    )
    def body(lat, vals, idx, dense, rb0, rb1, zbuf, lmax, smax, vstage,
             istage, sem0, sem1):
        iota = lax.iota(jnp.int32, 16)
        zero16 = jnp.zeros((16,), jnp.float32)
        wid = lax.axis_index("s") * 2 + lax.axis_index("c")
        base = wid * _RPW

        def zero_body(i, carry):
            zbuf[pl.ds(i * 16, 16)] = zero16
            return carry

        lax.fori_loop(0, DS // 16, zero_body, 0)

        def do_row(rbuf, row, rl):
            def p1(s, carry):
                off = s * 256
                mv = rbuf[pl.ds(off, 16)]
                for j in range(1, 16):
                    mv = jnp.maximum(mv, rbuf[pl.ds(off + j * 16, 16)])
                lmax[pl.ds(s * 16, 16)] = mv
                sb = (s // 16) * 16
                sv = smax[pl.ds(sb, 16)]
                smax[pl.ds(sb, 16)] = jnp.where(iota == s % 16,
                                                _rmax(mv), sv)
                return carry

            lax.fori_loop(0, _NSUP, p1, 0)

            def pop(i, carry):
                vv = list(carry[:4])
                ii = list(carry[4:])
                s0 = smax[pl.ds(0, 16)]
                s1 = smax[pl.ds(16, 16)]
                s2 = smax[pl.ds(32, 16)]
                s3 = smax[pl.ds(48, 16)]
                m = _rmax(jnp.maximum(jnp.maximum(s0, s1),
                                      jnp.maximum(s2, s3)))
                cand = jnp.int32(1 << 20)
                for jj, sv in enumerate((s0, s1, s2, s3)):
                    e = sv == m
                    pc = _scal(plsc.all_reduce_population_count(e))
                    ff = _scal(plsc.all_reduce_ffs(e))
                    cj = jnp.where(pc > 0, jj * 16 + ff, 1 << 20)
                    cand = jnp.minimum(cand, cj)
                su = cand
                lm = lmax[pl.ds(su * 16, 16)]
                lane = _scal(plsc.all_reduce_ffs(lm == m))
                idxv = su * 256 + lane + iota * 16
                d = plsc.load_gather(rbuf, [idxv])
                jpos = _scal(plsc.all_reduce_ffs(d == m))
                flat = su * 256 + jpos * 16 + lane
                qi = i // 16
                qr = i % 16
                for jj in range(4):
                    sel = jnp.logical_and(qi == jj, iota == qr)
                    vv[jj] = jnp.where(sel, m, vv[jj])
                    ii[jj] = jnp.where(sel, flat, ii[jj])
                d2 = jnp.where(iota == jpos, NEGINF, d)
                plsc.store_scatter(rbuf, [idxv], d2)
                cm = _rmax(d2)
                lm2 = jnp.where(iota == lane, cm, lm)
                lmax[pl.ds(su * 16, 16)] = lm2
                sb = (su // 16) * 16
                sv = smax[pl.ds(sb, 16)]
                smax[pl.ds(sb, 16)] = jnp.where(iota == su % 16,
                                                _rmax(lm2), sv)
                return tuple(vv) + tuple(ii)

            zero16i = jnp.zeros((16,), jnp.int32)
            init = (zero16, zero16, zero16, zero16,
                    zero16i, zero16i, zero16i, zero16i)
            res = lax.fori_loop(0, TOPK, pop, init)

            for jj in range(TOPK // 16):
                vstage[rl, pl.ds(jj * 16, 16)] = res[jj]
                istage[rl, pl.ds(jj * 16, 16)] = res[4 + jj]
                plsc.store_scatter(zbuf, [res[4 + jj]], res[jj])
            pltpu.sync_copy(zbuf, dense.at[row])
            for jj in range(TOPK // 16):
                plsc.store_scatter(zbuf, [res[4 + jj]], zero16)

        pltpu.async_copy(lat.at[base], rb0, sem0)

        def outer(o, carry):
            r = o * 2
            pltpu.async_copy(lat.at[base + r + 1], rb1, sem1)
            pltpu.make_async_copy(lat.at[base + r], rb0, sem0).wait()
            do_row(rb0, base + r, r)

            @pl.when(o < _RPW // 2 - 1)
            def _():
                pltpu.async_copy(lat.at[base + r + 2], rb0, sem0)

            pltpu.make_async_copy(lat.at[base + r + 1], rb1, sem1).wait()
            do_row(rb1, base + r + 1, r + 1)
            return carry

        lax.fori_loop(0, _RPW // 2, outer, 0)

        pltpu.sync_copy(vstage, vals.at[pl.ds(base, _RPW)])
        pltpu.sync_copy(istage, idx.at[pl.ds(base, _RPW)])

    return body(latents)


def kernel(x, W_enc, b_enc, W_dec, b_pre, k):
    del k
    latents = _encode(x, W_enc, b_enc, b_pre)
    vals, idxs, dense = _sc_topk(latents)
    x_hat = _decode(dense, W_dec.astype(jnp.bfloat16), b_pre)
    return x_hat, dense, idxs, vals
